# baseline (speedup 1.0000x reference)
import functools

import jax
import jax.numpy as jnp
from jax.experimental import pallas as pl
from jax.experimental.pallas import tpu as pltpu

_R = 16
_HALF = 112
_WP = 226


def _im2col(slab):
    left = pltpu.roll(slab, 1, axis=1)
    right = pltpu.roll(slab, _WP - 1, axis=1)
    return jnp.concatenate([left, slab, right], axis=0)


def _fused_body(x_ref, w1_ref, w2_ref, w3_ref, o_ref, xim0, xim1, xim2):
    R = _R
    s = pl.program_id(1)
    half = pl.program_id(0) % 2
    base = half * _HALF + s * R
    lane = jax.lax.broadcasted_iota(jnp.int32, (1, _WP), 1)
    lane_mask = ((lane >= 1) & (lane < _WP - 1)).astype(jnp.float32)

    def stage0(i, _):
        xim0[i] = _im2col(x_ref[0, 0, s * R + i])
        return _

    jax.lax.fori_loop(0, R + 6, stage0, None)

    def make_stage(w_ref, src, dst, off):
        def stage(i, _):
            rhs = src[pl.ds(i, 3)].reshape(3 * src.shape[1], _WP)
            y = jnp.dot(w_ref[...], rhs, preferred_element_type=jnp.float32)
            t = base + i - off
            valid = ((t >= 0) & (t < 2 * _HALF)).astype(jnp.float32)
            dst[i] = _im2col(y * (lane_mask * valid))
            return _
        return stage

    jax.lax.fori_loop(0, R + 4, make_stage(w1_ref, xim0, xim1, 2), None)
    jax.lax.fori_loop(0, R + 2, make_stage(w2_ref, xim1, xim2, 1), None)

    def stage3(i, _):
        rhs = xim2[pl.ds(i, 3)].reshape(3 * xim2.shape[1], _WP)
        o_ref[0, 0, 0, i] = jnp.dot(w3_ref[...], rhs,
                                    preferred_element_type=jnp.float32)
        return _

    jax.lax.fori_loop(0, R, stage3, None)


def kernel(x, w1, w2, w3):
    N, Ci, H, W = x.shape
    Cm = w1.shape[0]
    Co = w3.shape[0]
    R, HALF = _R, _HALF
    S = HALF // R

    xt = jnp.transpose(x, (0, 2, 1, 3))
    xp = jnp.pad(xt, ((0, 0), (3, 3), (0, 0), (1, 1)))
    xh = jnp.stack([xp[:, 0:HALF + 6], xp[:, HALF:2 * HALF + 6]], axis=1)

    def wmat(w):
        return jnp.transpose(w, (0, 2, 3, 1)).reshape(w.shape[0], -1)

    w1m, w2m, w3m = wmat(w1), wmat(w2), wmat(w3)

    out = pl.pallas_call(
        _fused_body,
        grid=(2 * N, S),
        in_specs=[
            pl.BlockSpec((1, 1, HALF + 6, Ci, _WP),
                         lambda n2, s: (n2 // 2, n2 % 2, 0, 0, 0)),
            pl.BlockSpec(w1m.shape, lambda n2, s: (0, 0)),
            pl.BlockSpec(w2m.shape, lambda n2, s: (0, 0)),
            pl.BlockSpec(w3m.shape, lambda n2, s: (0, 0)),
        ],
        out_specs=pl.BlockSpec((1, 1, 1, R, Co, _WP),
                               lambda n2, s: (n2 // 2, n2 % 2, s, 0, 0, 0)),
        out_shape=jax.ShapeDtypeStruct((N, 2, S, R, Co, _WP), jnp.float32),
        scratch_shapes=[
            pltpu.VMEM((R + 6, 3 * Ci, _WP), jnp.float32),
            pltpu.VMEM((R + 4, 3 * Cm, _WP), jnp.float32),
            pltpu.VMEM((R + 2, 3 * Cm, _WP), jnp.float32),
        ],
        compiler_params=pltpu.CompilerParams(
            dimension_semantics=("parallel", "arbitrary"),
            vmem_limit_bytes=56 * 1024 * 1024,
        ),
        name="fused_conv3",
    )(xh, w1m, w2m, w3m)

    y = out.reshape(N, H, Co, _WP)[:, :, :, 1:W + 1]
    return jnp.transpose(y, (0, 2, 1, 3))


# speedup vs baseline: 1.4853x; 1.4853x over previous
import functools

import jax
import jax.numpy as jnp
from jax.experimental import pallas as pl
from jax.experimental.pallas import tpu as pltpu

_R = 16
_HALF = 112
_WP = 226


def _im2col(slab):
    left = pltpu.roll(slab, 1, axis=1)
    right = pltpu.roll(slab, _WP - 1, axis=1)
    return jnp.concatenate([left, slab, right], axis=0)


def _fused_body(x_ref, w1_ref, w2_ref, w3_ref, o_ref, xim0, xim1, xim2):
    R = _R
    s = pl.program_id(1)
    half = pl.program_id(0) % 2
    base = half * _HALF + s * R
    lane = jax.lax.broadcasted_iota(jnp.int32, (1, _WP), 1)
    lane_mask = ((lane >= 1) & (lane < _WP - 1)).astype(jnp.float32)

    for i in range(R + 6):
        xim0[i] = _im2col(x_ref[0, 0, s * R + i])

    def stage(w_ref, src, dst, off, i):
        rhs = src[pl.ds(i, 3)].reshape(3 * src.shape[1], _WP)
        y = jnp.dot(w_ref[...], rhs, preferred_element_type=jnp.float32)
        t = base + i - off
        valid = ((t >= 0) & (t < 2 * _HALF)).astype(jnp.float32)
        dst[i] = _im2col(y * (lane_mask * valid))

    for i in range(R + 4):
        stage(w1_ref, xim0, xim1, 2, i)
    for i in range(R + 2):
        stage(w2_ref, xim1, xim2, 1, i)

    for i in range(R):
        rhs = xim2[pl.ds(i, 3)].reshape(3 * xim2.shape[1], _WP)
        o_ref[0, 0, 0, i] = jnp.dot(w3_ref[...], rhs,
                                    preferred_element_type=jnp.float32)


def kernel(x, w1, w2, w3):
    N, Ci, H, W = x.shape
    Cm = w1.shape[0]
    Co = w3.shape[0]
    R, HALF = _R, _HALF
    S = HALF // R

    xt = jnp.transpose(x, (0, 2, 1, 3))
    xp = jnp.pad(xt, ((0, 0), (3, 3), (0, 0), (1, 1)))
    xh = jnp.stack([xp[:, 0:HALF + 6], xp[:, HALF:2 * HALF + 6]], axis=1)

    def wmat(w):
        return jnp.transpose(w, (0, 2, 3, 1)).reshape(w.shape[0], -1)

    w1m, w2m, w3m = wmat(w1), wmat(w2), wmat(w3)

    out = pl.pallas_call(
        _fused_body,
        grid=(2 * N, S),
        in_specs=[
            pl.BlockSpec((1, 1, HALF + 6, Ci, _WP),
                         lambda n2, s: (n2 // 2, n2 % 2, 0, 0, 0)),
            pl.BlockSpec(w1m.shape, lambda n2, s: (0, 0)),
            pl.BlockSpec(w2m.shape, lambda n2, s: (0, 0)),
            pl.BlockSpec(w3m.shape, lambda n2, s: (0, 0)),
        ],
        out_specs=pl.BlockSpec((1, 1, 1, R, Co, _WP),
                               lambda n2, s: (n2 // 2, n2 % 2, s, 0, 0, 0)),
        out_shape=jax.ShapeDtypeStruct((N, 2, S, R, Co, _WP), jnp.float32),
        scratch_shapes=[
            pltpu.VMEM((R + 6, 3 * Ci, _WP), jnp.float32),
            pltpu.VMEM((R + 4, 3 * Cm, _WP), jnp.float32),
            pltpu.VMEM((R + 2, 3 * Cm, _WP), jnp.float32),
        ],
        compiler_params=pltpu.CompilerParams(
            dimension_semantics=("parallel", "arbitrary"),
            vmem_limit_bytes=56 * 1024 * 1024,
        ),
        name="fused_conv3",
    )(xh, w1m, w2m, w3m)

    y = out.reshape(N, H, Co, _WP)[:, :, :, 1:W + 1]
    return jnp.transpose(y, (0, 2, 1, 3))


# speedup vs baseline: 3.3820x; 2.2769x over previous
import functools

import jax
import jax.numpy as jnp
from jax.experimental import pallas as pl
from jax.experimental.pallas import tpu as pltpu

_R = 16
_HALF = 112
_WP = 226


def _im2col(slab):
    left = jnp.concatenate([slab[:, _WP - 1:], slab[:, :_WP - 1]], axis=1)
    right = jnp.concatenate([slab[:, 1:], slab[:, :1]], axis=1)
    return jnp.concatenate([left, slab, right], axis=0)


def _fused_body(x_ref, w1_ref, w2_ref, w3_ref, o_ref, xim0, xim1, xim2):
    R = _R
    s = pl.program_id(1)
    half = pl.program_id(0) % 2
    base = half * _HALF + s * R
    lane = jax.lax.broadcasted_iota(jnp.int32, (1, _WP), 1)
    lane_mask = ((lane >= 1) & (lane < _WP - 1)).astype(jnp.float32)

    for i in range(R + 6):
        xim0[i] = _im2col(x_ref[0, 0, s * R + i])

    def stage(w_ref, src, dst, off, i):
        rhs = src[pl.ds(i, 3)].reshape(3 * src.shape[1], _WP)
        y = jnp.dot(w_ref[...], rhs, preferred_element_type=jnp.float32)
        t = base + i - off
        valid = ((t >= 0) & (t < 2 * _HALF)).astype(jnp.float32)
        dst[i] = _im2col(y * (lane_mask * valid))

    for i in range(R + 4):
        stage(w1_ref, xim0, xim1, 2, i)
    for i in range(R + 2):
        stage(w2_ref, xim1, xim2, 1, i)

    for i in range(R):
        rhs = xim2[pl.ds(i, 3)].reshape(3 * xim2.shape[1], _WP)
        o_ref[0, 0, 0, i] = jnp.dot(w3_ref[...], rhs,
                                    preferred_element_type=jnp.float32)


def kernel(x, w1, w2, w3):
    N, Ci, H, W = x.shape
    Cm = w1.shape[0]
    Co = w3.shape[0]
    R, HALF = _R, _HALF
    S = HALF // R

    xt = jnp.transpose(x, (0, 2, 1, 3))
    xp = jnp.pad(xt, ((0, 0), (3, 3), (0, 0), (1, 1)))
    xh = jnp.stack([xp[:, 0:HALF + 6], xp[:, HALF:2 * HALF + 6]], axis=1)

    def wmat(w):
        return jnp.transpose(w, (0, 2, 3, 1)).reshape(w.shape[0], -1)

    w1m, w2m, w3m = wmat(w1), wmat(w2), wmat(w3)

    out = pl.pallas_call(
        _fused_body,
        grid=(2 * N, S),
        in_specs=[
            pl.BlockSpec((1, 1, HALF + 6, Ci, _WP),
                         lambda n2, s: (n2 // 2, n2 % 2, 0, 0, 0)),
            pl.BlockSpec(w1m.shape, lambda n2, s: (0, 0)),
            pl.BlockSpec(w2m.shape, lambda n2, s: (0, 0)),
            pl.BlockSpec(w3m.shape, lambda n2, s: (0, 0)),
        ],
        out_specs=pl.BlockSpec((1, 1, 1, R, Co, _WP),
                               lambda n2, s: (n2 // 2, n2 % 2, s, 0, 0, 0)),
        out_shape=jax.ShapeDtypeStruct((N, 2, S, R, Co, _WP), jnp.float32),
        scratch_shapes=[
            pltpu.VMEM((R + 6, 3 * Ci, _WP), jnp.float32),
            pltpu.VMEM((R + 4, 3 * Cm, _WP), jnp.float32),
            pltpu.VMEM((R + 2, 3 * Cm, _WP), jnp.float32),
        ],
        compiler_params=pltpu.CompilerParams(
            dimension_semantics=("parallel", "arbitrary"),
            vmem_limit_bytes=56 * 1024 * 1024,
        ),
        name="fused_conv3",
    )(xh, w1m, w2m, w3m)

    y = out.reshape(N, H, Co, _WP)[:, :, :, 1:W + 1]
    return jnp.transpose(y, (0, 2, 1, 3))


# speedup vs baseline: 4.8293x; 1.4280x over previous
import jax
import jax.numpy as jnp
from jax.experimental import pallas as pl
from jax.experimental.pallas import tpu as pltpu

_R = 28
_HALF = 112
_WP = 226


def _im2col(slab):
    left = jnp.concatenate([slab[:, _WP - 1:], slab[:, :_WP - 1]], axis=1)
    right = jnp.concatenate([slab[:, 1:], slab[:, :1]], axis=1)
    return jnp.concatenate([left, slab, right], axis=0)


def _fused_body(x_ref, w1_ref, w2_ref, w3_ref, o_ref, xim0, xim1, xim2):
    R = _R
    s = pl.program_id(1)
    half = pl.program_id(0) % 2
    base = half * _HALF + s * R
    lane = jax.lax.broadcasted_iota(jnp.int32, (1, _WP), 1)
    lane_mask = ((lane >= 1) & (lane < _WP - 1)).astype(jnp.float32)

    for i in range(R + 6):
        xim0[i] = _im2col(x_ref[0, 0, s * R + i])

    def stage(w_ref, src, dst, off, i):
        rhs = src[pl.ds(i, 3)].reshape(3 * src.shape[1], _WP)
        y = jnp.dot(w_ref[...], rhs, preferred_element_type=jnp.float32)
        t = base + i - off
        valid = ((t >= 0) & (t < 2 * _HALF)).astype(jnp.float32)
        dst[i] = _im2col((y * (lane_mask * valid)).astype(jnp.bfloat16))

    for i in range(R + 4):
        stage(w1_ref, xim0, xim1, 2, i)
    for i in range(R + 2):
        stage(w2_ref, xim1, xim2, 1, i)

    for i in range(R):
        rhs = xim2[pl.ds(i, 3)].reshape(3 * xim2.shape[1], _WP)
        o_ref[0, 0, 0, i] = jnp.dot(w3_ref[...], rhs,
                                    preferred_element_type=jnp.float32)


def kernel(x, w1, w2, w3):
    N, Ci, H, W = x.shape
    Cm = w1.shape[0]
    Co = w3.shape[0]
    R, HALF = _R, _HALF
    S = HALF // R

    xt = jnp.transpose(x, (0, 2, 1, 3)).astype(jnp.bfloat16)
    xp = jnp.pad(xt, ((0, 0), (3, 3), (0, 0), (1, 1)))
    xh = jnp.stack([xp[:, 0:HALF + 6], xp[:, HALF:2 * HALF + 6]], axis=1)

    def wmat(w):
        m = jnp.transpose(w, (0, 2, 3, 1)).reshape(w.shape[0], -1)
        return m.astype(jnp.bfloat16)

    w1m, w2m, w3m = wmat(w1), wmat(w2), wmat(w3)

    out = pl.pallas_call(
        _fused_body,
        grid=(2 * N, S),
        in_specs=[
            pl.BlockSpec((1, 1, HALF + 6, Ci, _WP),
                         lambda n2, s: (n2 // 2, n2 % 2, 0, 0, 0)),
            pl.BlockSpec(w1m.shape, lambda n2, s: (0, 0)),
            pl.BlockSpec(w2m.shape, lambda n2, s: (0, 0)),
            pl.BlockSpec(w3m.shape, lambda n2, s: (0, 0)),
        ],
        out_specs=pl.BlockSpec((1, 1, 1, R, Co, _WP),
                               lambda n2, s: (n2 // 2, n2 % 2, s, 0, 0, 0)),
        out_shape=jax.ShapeDtypeStruct((N, 2, S, R, Co, _WP), jnp.float32),
        scratch_shapes=[
            pltpu.VMEM((R + 6, 3 * Ci, _WP), jnp.bfloat16),
            pltpu.VMEM((R + 4, 3 * Cm, _WP), jnp.bfloat16),
            pltpu.VMEM((R + 2, 3 * Cm, _WP), jnp.bfloat16),
        ],
        compiler_params=pltpu.CompilerParams(
            dimension_semantics=("parallel", "arbitrary"),
            vmem_limit_bytes=56 * 1024 * 1024,
        ),
        name="fused_conv3",
    )(xh, w1m, w2m, w3m)

    y = out.reshape(N, H, Co, _WP)[:, :, :, 1:W + 1]
    return jnp.transpose(y, (0, 2, 1, 3))


# speedup vs baseline: 5.0664x; 1.0491x over previous
import jax
import jax.numpy as jnp
from jax.experimental import pallas as pl
from jax.experimental.pallas import tpu as pltpu

_R = 28
_WP = 226


def _im2col(slab):
    left = jnp.concatenate([slab[:, _WP - 1:], slab[:, :_WP - 1]], axis=1)
    right = jnp.concatenate([slab[:, 1:], slab[:, :1]], axis=1)
    return jnp.concatenate([left, slab, right], axis=0)


def _fused_body(x_ref, w1_ref, w2_ref, w3_ref, o_ref, xim0, xim1, xim2):
    R = _R
    s = pl.program_id(1)
    base = s * R
    lane = jax.lax.broadcasted_iota(jnp.int32, (1, _WP), 1)
    lane_mask = ((lane >= 1) & (lane < _WP - 1)).astype(jnp.float32)

    for i in range(R + 6):
        xim0[i] = _im2col(x_ref[0, s * R + i])

    def stage(w_ref, src, dst, off, i):
        rhs = src[pl.ds(i, 3)].reshape(3 * src.shape[1], _WP)
        y = jnp.dot(w_ref[...], rhs, preferred_element_type=jnp.float32)
        t = base + i - off
        valid = ((t >= 0) & (t < 224)).astype(jnp.float32)
        dst[i] = _im2col((y * (lane_mask * valid)).astype(jnp.bfloat16))

    for i in range(R + 4):
        stage(w1_ref, xim0, xim1, 2, i)
    for i in range(R + 2):
        stage(w2_ref, xim1, xim2, 1, i)

    for i in range(R):
        rhs = xim2[pl.ds(i, 3)].reshape(3 * xim2.shape[1], _WP)
        o_ref[0, 0, i] = jnp.dot(w3_ref[...], rhs,
                                 preferred_element_type=jnp.float32)


def kernel(x, w1, w2, w3):
    N, Ci, H, W = x.shape
    Cm = w1.shape[0]
    Co = w3.shape[0]
    R = _R
    S = H // R

    xt = jnp.transpose(x, (0, 2, 1, 3)).astype(jnp.bfloat16)
    xp = jnp.pad(xt, ((0, 0), (3, 3), (0, 0), (1, 1)))

    def wmat(w):
        m = jnp.transpose(w, (0, 2, 3, 1)).reshape(w.shape[0], -1)
        return m.astype(jnp.bfloat16)

    w1m, w2m, w3m = wmat(w1), wmat(w2), wmat(w3)

    out = pl.pallas_call(
        _fused_body,
        grid=(N, S),
        in_specs=[
            pl.BlockSpec((1, H + 6, Ci, _WP),
                         lambda n, s: (n, 0, 0, 0)),
            pl.BlockSpec(w1m.shape, lambda n, s: (0, 0)),
            pl.BlockSpec(w2m.shape, lambda n, s: (0, 0)),
            pl.BlockSpec(w3m.shape, lambda n, s: (0, 0)),
        ],
        out_specs=pl.BlockSpec((1, 1, R, Co, _WP),
                               lambda n, s: (n, s, 0, 0, 0)),
        out_shape=jax.ShapeDtypeStruct((N, S, R, Co, _WP), jnp.float32),
        scratch_shapes=[
            pltpu.VMEM((R + 6, 3 * Ci, _WP), jnp.bfloat16),
            pltpu.VMEM((R + 4, 3 * Cm, _WP), jnp.bfloat16),
            pltpu.VMEM((R + 2, 3 * Cm, _WP), jnp.bfloat16),
        ],
        compiler_params=pltpu.CompilerParams(
            dimension_semantics=("parallel", "arbitrary"),
            vmem_limit_bytes=56 * 1024 * 1024,
        ),
        name="fused_conv3",
    )(xp, w1m, w2m, w3m)

    y = out.reshape(N, H, Co, _WP)[:, :, :, 1:W + 1]
    return jnp.transpose(y, (0, 2, 1, 3))


# speedup vs baseline: 5.5665x; 1.0987x over previous
import jax
import jax.numpy as jnp
from jax.experimental import pallas as pl
from jax.experimental.pallas import tpu as pltpu

_R = 28
_WP = 226


def _im2col(slab):
    left = jnp.concatenate([slab[:, _WP - 1:], slab[:, :_WP - 1]], axis=1)
    right = jnp.concatenate([slab[:, 1:], slab[:, :1]], axis=1)
    return jnp.concatenate([left, slab, right], axis=0)


def _fused_body(x_ref, w1_ref, w2_ref, w3_ref, o_ref, xim0, xim1, xim2):
    R = _R
    s = pl.program_id(1)
    base = s * R
    lane = jax.lax.broadcasted_iota(jnp.int32, (1, _WP), 1)
    lane_mask = ((lane >= 1) & (lane < _WP - 1)).astype(jnp.float32)

    for i in range(R + 6):
        xim0[i] = _im2col(x_ref[0, s * R + i])

    def stage(w_ref, src, dst, off, i):
        rhs = src[pl.ds(i, 3)].reshape(3 * src.shape[1], _WP)
        y = jnp.dot(w_ref[...], rhs, preferred_element_type=jnp.float32)
        t = base + i - off
        valid = ((t >= 0) & (t < 224)).astype(jnp.float32)
        dst[i] = _im2col((y * (lane_mask * valid)).astype(jnp.bfloat16))

    for i in range(R + 4):
        stage(w1_ref, xim0, xim1, 2, i)
    for i in range(R + 2):
        stage(w2_ref, xim1, xim2, 1, i)

    for i in range(R):
        rhs = xim2[pl.ds(i, 3)].reshape(3 * xim2.shape[1], _WP)
        y3 = jnp.dot(w3_ref[...], rhs, preferred_element_type=jnp.float32)
        o_ref[0, 0, i] = y3.astype(jnp.bfloat16)


def kernel(x, w1, w2, w3):
    N, Ci, H, W = x.shape
    Cm = w1.shape[0]
    Co = w3.shape[0]
    R = _R
    S = H // R

    xt = jnp.transpose(x, (0, 2, 1, 3)).astype(jnp.bfloat16)
    xp = jnp.pad(xt, ((0, 0), (3, 3), (0, 0), (1, 1)))

    def wmat(w):
        m = jnp.transpose(w, (0, 2, 3, 1)).reshape(w.shape[0], -1)
        return m.astype(jnp.bfloat16)

    w1m, w2m, w3m = wmat(w1), wmat(w2), wmat(w3)

    out = pl.pallas_call(
        _fused_body,
        grid=(N, S),
        in_specs=[
            pl.BlockSpec((1, H + 6, Ci, _WP),
                         lambda n, s: (n, 0, 0, 0)),
            pl.BlockSpec(w1m.shape, lambda n, s: (0, 0)),
            pl.BlockSpec(w2m.shape, lambda n, s: (0, 0)),
            pl.BlockSpec(w3m.shape, lambda n, s: (0, 0)),
        ],
        out_specs=pl.BlockSpec((1, 1, R, Co, _WP),
                               lambda n, s: (n, s, 0, 0, 0)),
        out_shape=jax.ShapeDtypeStruct((N, S, R, Co, _WP), jnp.bfloat16),
        scratch_shapes=[
            pltpu.VMEM((R + 6, 3 * Ci, _WP), jnp.bfloat16),
            pltpu.VMEM((R + 4, 3 * Cm, _WP), jnp.bfloat16),
            pltpu.VMEM((R + 2, 3 * Cm, _WP), jnp.bfloat16),
        ],
        compiler_params=pltpu.CompilerParams(
            dimension_semantics=("parallel", "arbitrary"),
            vmem_limit_bytes=56 * 1024 * 1024,
        ),
        name="fused_conv3",
    )(xp, w1m, w2m, w3m)

    y = out.reshape(N, H, Co, _WP)[:, :, :, 1:W + 1].astype(jnp.float32)
    return jnp.transpose(y, (0, 2, 1, 3))


# speedup vs baseline: 5.7730x; 1.0371x over previous
import jax
import jax.numpy as jnp
from jax.experimental import pallas as pl
from jax.experimental.pallas import tpu as pltpu

_R = 56
_WP = 226


def _im2col(slab):
    left = jnp.concatenate([slab[:, _WP - 1:], slab[:, :_WP - 1]], axis=1)
    right = jnp.concatenate([slab[:, 1:], slab[:, :1]], axis=1)
    return jnp.concatenate([left, slab, right], axis=0)


def _fused_body(x_ref, w1_ref, w2_ref, w3_ref, o_ref, xim0, xim1, xim2):
    R = _R
    s = pl.program_id(1)
    base = s * R
    lane = jax.lax.broadcasted_iota(jnp.int32, (1, _WP), 1)
    lane_mask = ((lane >= 1) & (lane < _WP - 1)).astype(jnp.float32)

    for i in range(R + 6):
        xim0[i] = _im2col(x_ref[0, s * R + i])

    def stage(w_ref, src, dst, off, i):
        rhs = src[pl.ds(i, 3)].reshape(3 * src.shape[1], _WP)
        y = jnp.dot(w_ref[...], rhs, preferred_element_type=jnp.float32)
        t = base + i - off
        valid = ((t >= 0) & (t < 224)).astype(jnp.float32)
        dst[i] = _im2col((y * (lane_mask * valid)).astype(jnp.bfloat16))

    for i in range(R + 4):
        stage(w1_ref, xim0, xim1, 2, i)
    for i in range(R + 2):
        stage(w2_ref, xim1, xim2, 1, i)

    for i in range(R):
        rhs = xim2[pl.ds(i, 3)].reshape(3 * xim2.shape[1], _WP)
        y3 = jnp.dot(w3_ref[...], rhs, preferred_element_type=jnp.float32)
        o_ref[0, 0, i] = y3.astype(jnp.bfloat16)


def kernel(x, w1, w2, w3):
    N, Ci, H, W = x.shape
    Cm = w1.shape[0]
    Co = w3.shape[0]
    R = _R
    S = H // R

    xt = jnp.transpose(x, (0, 2, 1, 3)).astype(jnp.bfloat16)
    xp = jnp.pad(xt, ((0, 0), (3, 3), (0, 0), (1, 1)))

    def wmat(w):
        m = jnp.transpose(w, (0, 2, 3, 1)).reshape(w.shape[0], -1)
        return m.astype(jnp.bfloat16)

    w1m, w2m, w3m = wmat(w1), wmat(w2), wmat(w3)

    out = pl.pallas_call(
        _fused_body,
        grid=(N, S),
        in_specs=[
            pl.BlockSpec((1, H + 6, Ci, _WP),
                         lambda n, s: (n, 0, 0, 0)),
            pl.BlockSpec(w1m.shape, lambda n, s: (0, 0)),
            pl.BlockSpec(w2m.shape, lambda n, s: (0, 0)),
            pl.BlockSpec(w3m.shape, lambda n, s: (0, 0)),
        ],
        out_specs=pl.BlockSpec((1, 1, R, Co, _WP),
                               lambda n, s: (n, s, 0, 0, 0)),
        out_shape=jax.ShapeDtypeStruct((N, S, R, Co, _WP), jnp.bfloat16),
        scratch_shapes=[
            pltpu.VMEM((R + 6, 3 * Ci, _WP), jnp.bfloat16),
            pltpu.VMEM((R + 4, 3 * Cm, _WP), jnp.bfloat16),
            pltpu.VMEM((R + 2, 3 * Cm, _WP), jnp.bfloat16),
        ],
        compiler_params=pltpu.CompilerParams(
            dimension_semantics=("parallel", "arbitrary"),
            vmem_limit_bytes=56 * 1024 * 1024,
        ),
        name="fused_conv3",
    )(xp, w1m, w2m, w3m)

    y = out.reshape(N, H, Co, _WP)[:, :, :, 1:W + 1].astype(jnp.float32)
    return jnp.transpose(y, (0, 2, 1, 3))


# speedup vs baseline: 6.5595x; 1.1362x over previous
import jax
import jax.numpy as jnp
from jax.experimental import pallas as pl
from jax.experimental.pallas import tpu as pltpu

_R = 56
_WP = 226


def _im2col(slab):
    left = jnp.concatenate([slab[:, _WP - 1:], slab[:, :_WP - 1]], axis=1)
    right = jnp.concatenate([slab[:, 1:], slab[:, :1]], axis=1)
    return jnp.concatenate([left, slab, right], axis=0)


def _fused_body(x_ref, w1_ref, w2_ref, w3_ref, o_ref, xim0, xim1, xim2):
    R = _R
    s = pl.program_id(1)
    base = s * R
    lane = jax.lax.broadcasted_iota(jnp.int32, (1, _WP), 1)
    lane_mask = ((lane >= 1) & (lane < _WP - 1)).astype(jnp.float32)

    for i in range(R + 6):
        xim0[i] = _im2col(x_ref[0, s * R + i])

    def stage(w_ref, src, dst, off, i):
        rhs = src[pl.ds(i, 3)].reshape(3 * src.shape[1], _WP)
        y = jnp.dot(w_ref[...], rhs, preferred_element_type=jnp.float32)
        t = base + i - off
        valid = ((t >= 0) & (t < 224)).astype(jnp.float32)
        dst[i] = _im2col((y * (lane_mask * valid)).astype(jnp.bfloat16))

    for i in range(R + 4):
        stage(w1_ref, xim0, xim1, 2, i)
    for i in range(R + 2):
        stage(w2_ref, xim1, xim2, 1, i)

    for i in range(R):
        rhs = xim2[pl.ds(i, 3)].reshape(3 * xim2.shape[1], _WP)
        y3 = jnp.dot(w3_ref[...], rhs, preferred_element_type=jnp.float32)
        o_ref[0, :, i, :] = y3[:, 1:_WP - 1]


def kernel(x, w1, w2, w3):
    N, Ci, H, W = x.shape
    Cm = w1.shape[0]
    Co = w3.shape[0]
    R = _R
    S = H // R

    xt = jnp.transpose(x, (0, 2, 1, 3)).astype(jnp.bfloat16)
    xp = jnp.pad(xt, ((0, 0), (3, 3), (0, 0), (1, 1)))

    def wmat(w):
        m = jnp.transpose(w, (0, 2, 3, 1)).reshape(w.shape[0], -1)
        return m.astype(jnp.bfloat16)

    w1m, w2m, w3m = wmat(w1), wmat(w2), wmat(w3)

    out = pl.pallas_call(
        _fused_body,
        grid=(N, S),
        in_specs=[
            pl.BlockSpec((1, H + 6, Ci, _WP),
                         lambda n, s: (n, 0, 0, 0)),
            pl.BlockSpec(w1m.shape, lambda n, s: (0, 0)),
            pl.BlockSpec(w2m.shape, lambda n, s: (0, 0)),
            pl.BlockSpec(w3m.shape, lambda n, s: (0, 0)),
        ],
        out_specs=pl.BlockSpec((1, Co, R, W),
                               lambda n, s: (n, 0, s, 0)),
        out_shape=jax.ShapeDtypeStruct((N, Co, H, W), jnp.float32),
        scratch_shapes=[
            pltpu.VMEM((R + 6, 3 * Ci, _WP), jnp.bfloat16),
            pltpu.VMEM((R + 4, 3 * Cm, _WP), jnp.bfloat16),
            pltpu.VMEM((R + 2, 3 * Cm, _WP), jnp.bfloat16),
        ],
        compiler_params=pltpu.CompilerParams(
            dimension_semantics=("parallel", "arbitrary"),
            vmem_limit_bytes=56 * 1024 * 1024,
        ),
        name="fused_conv3",
    )(xp, w1m, w2m, w3m)

    return out


# speedup vs baseline: 6.9487x; 1.0593x over previous
import jax
import jax.numpy as jnp
from jax.experimental import pallas as pl
from jax.experimental.pallas import tpu as pltpu

_R = 32
_WP = 226


def _im2col(slab):
    left = jnp.concatenate([slab[:, _WP - 1:], slab[:, :_WP - 1]], axis=1)
    right = jnp.concatenate([slab[:, 1:], slab[:, :1]], axis=1)
    return jnp.concatenate([left, slab, right], axis=0)


def _fused_body(x_ref, w1_ref, w2_ref, w3_ref, o_ref, xim0, xim1, xim2):
    R = _R
    s = pl.program_id(1)
    base = s * R
    lane = jax.lax.broadcasted_iota(jnp.int32, (1, _WP), 1)
    lane_mask = ((lane >= 1) & (lane < _WP - 1)).astype(jnp.float32)

    z1 = jnp.zeros((x_ref.shape[1], 1), jnp.float32)
    for i in range(R + 6):
        t = base + i - 3
        tc = jnp.clip(t, 0, 223)
        row = x_ref[0, :, tc, :]
        v = ((t >= 0) & (t < 224)).astype(jnp.float32)
        slab = jnp.concatenate([z1, row * v, z1], axis=1)
        xim0[i] = _im2col(slab.astype(jnp.bfloat16))

    def stage(w_ref, src, dst, off, i):
        rhs = src[pl.ds(i, 3)].reshape(3 * src.shape[1], _WP)
        y = jnp.dot(w_ref[...], rhs, preferred_element_type=jnp.float32)
        t = base + i - off
        valid = ((t >= 0) & (t < 224)).astype(jnp.float32)
        dst[i] = _im2col((y * (lane_mask * valid)).astype(jnp.bfloat16))

    for i in range(R + 4):
        stage(w1_ref, xim0, xim1, 2, i)
    for i in range(R + 2):
        stage(w2_ref, xim1, xim2, 1, i)

    for i in range(R):
        rhs = xim2[pl.ds(i, 3)].reshape(3 * xim2.shape[1], _WP)
        y3 = jnp.dot(w3_ref[...], rhs, preferred_element_type=jnp.float32)
        o_ref[0, :, i, :] = y3[:, 1:_WP - 1]


def kernel(x, w1, w2, w3):
    N, Ci, H, W = x.shape
    Cm = w1.shape[0]
    Co = w3.shape[0]
    R = _R
    S = H // R

    def wmat(w):
        m = jnp.transpose(w, (0, 2, 3, 1)).reshape(w.shape[0], -1)
        return m.astype(jnp.bfloat16)

    w1m, w2m, w3m = wmat(w1), wmat(w2), wmat(w3)

    out = pl.pallas_call(
        _fused_body,
        grid=(N, S),
        in_specs=[
            pl.BlockSpec((1, Ci, H, W),
                         lambda n, s: (n, 0, 0, 0)),
            pl.BlockSpec(w1m.shape, lambda n, s: (0, 0)),
            pl.BlockSpec(w2m.shape, lambda n, s: (0, 0)),
            pl.BlockSpec(w3m.shape, lambda n, s: (0, 0)),
        ],
        out_specs=pl.BlockSpec((1, Co, R, W),
                               lambda n, s: (n, 0, s, 0)),
        out_shape=jax.ShapeDtypeStruct((N, Co, H, W), jnp.float32),
        scratch_shapes=[
            pltpu.VMEM((R + 6, 3 * Ci, _WP), jnp.bfloat16),
            pltpu.VMEM((R + 4, 3 * Cm, _WP), jnp.bfloat16),
            pltpu.VMEM((R + 2, 3 * Cm, _WP), jnp.bfloat16),
        ],
        compiler_params=pltpu.CompilerParams(
            dimension_semantics=("parallel", "arbitrary"),
            vmem_limit_bytes=56 * 1024 * 1024,
        ),
        name="fused_conv3",
    )(x, w1m, w2m, w3m)

    return out
